# Initial kernel scaffold; baseline (speedup 1.0000x reference)
import jax
import jax.numpy as jnp
from jax.experimental import pallas as pl


def kernel(means2d, conics, colors, opacities, depths, background, image_height, image_width):
    raise NotImplementedError("write your pallas kernel here")


# speedup vs baseline: 56.1175x; 56.1175x over previous
import jax
import jax.numpy as jnp
from jax.experimental import pallas as pl
from jax.experimental.pallas import tpu as pltpu

ALPHA_THRESHOLD = 1.0 / 255.0
MAX_ALPHA = 0.99
EPS = 1e-8
PIX_OFF = 0.5
H = 128
W = 128
G = 2048
K = 8
NCHUNK = G // K


def _raster_body(params_ref, bg_ref, out_ref, accr, accg, accb, trans_ref):
    i = pl.program_id(0)

    @pl.when(i == 0)
    def _init():
        accr[:, :] = jnp.zeros((H, W), jnp.float32)
        accg[:, :] = jnp.zeros((H, W), jnp.float32)
        accb[:, :] = jnp.zeros((H, W), jnp.float32)
        trans_ref[:, :] = jnp.ones((H, W), jnp.float32)

    p = params_ref[:, :]
    mx = p[:, 0:1][:, :, None]
    my = p[:, 1:2][:, :, None]
    a = p[:, 2:3][:, :, None]
    b = p[:, 3:4][:, :, None]
    c = p[:, 4:5][:, :, None]
    op = p[:, 5:6][:, :, None]

    det = a * c - b * b
    tau = -2.0 * jnp.log(jnp.maximum(ALPHA_THRESHOLD / jnp.maximum(op, EPS), EPS))
    valid = (op > ALPHA_THRESHOLD) & (det > EPS) & (a > 0.0) & (c > 0.0) & (tau > 0.0)

    xs = jax.lax.broadcasted_iota(jnp.int32, (1, 1, W), 2).astype(jnp.float32) + PIX_OFF
    ys = jax.lax.broadcasted_iota(jnp.int32, (1, H, 1), 1).astype(jnp.float32) + PIX_OFF
    dx = xs - mx
    dy = ys - my
    q = a * (dx * dx) + 2.0 * b * (dx * dy) + c * (dy * dy)
    alpha = jnp.where((q <= tau) & valid, op * jnp.exp(-0.5 * q), 0.0)
    alpha = jnp.minimum(alpha, MAX_ALPHA)

    t = trans_ref[:, :]
    ws = []
    for g in range(K):
        ag = alpha[g]
        ws.append(t * ag)
        t = t * (1.0 - ag)
    wstack = jnp.stack(ws, axis=0)

    cr = p[:, 6:7][:, :, None]
    cg = p[:, 7:8][:, :, None]
    cb = p[:, 8:9][:, :, None]
    accr[:, :] += jnp.sum(wstack * cr, axis=0)
    accg[:, :] += jnp.sum(wstack * cg, axis=0)
    accb[:, :] += jnp.sum(wstack * cb, axis=0)
    trans_ref[:, :] = t

    @pl.when(i == NCHUNK - 1)
    def _fin():
        tt = trans_ref[:, :]
        out_ref[0, :, :] = accr[:, :] + tt * bg_ref[0]
        out_ref[1, :, :] = accg[:, :] + tt * bg_ref[1]
        out_ref[2, :, :] = accb[:, :] + tt * bg_ref[2]


def kernel(means2d, conics, colors, opacities, depths, background, image_height, image_width):
    order = jnp.argsort(jax.lax.stop_gradient(depths))
    params = jnp.zeros((G, 16), jnp.float32)
    params = params.at[:, 0:2].set(means2d)
    params = params.at[:, 2:5].set(conics)
    params = params.at[:, 5].set(opacities)
    params = params.at[:, 6:9].set(colors)
    params = jnp.take(params, order, axis=0)

    out = pl.pallas_call(
        _raster_body,
        grid=(NCHUNK,),
        in_specs=[
            pl.BlockSpec((K, 16), lambda i: (i, 0)),
            pl.BlockSpec(memory_space=pltpu.SMEM),
        ],
        out_specs=pl.BlockSpec((3, H, W), lambda i: (0, 0, 0)),
        out_shape=jax.ShapeDtypeStruct((3, H, W), jnp.float32),
        scratch_shapes=[
            pltpu.VMEM((H, W), jnp.float32),
            pltpu.VMEM((H, W), jnp.float32),
            pltpu.VMEM((H, W), jnp.float32),
            pltpu.VMEM((H, W), jnp.float32),
        ],
    )(params, background.astype(jnp.float32))
    return jnp.transpose(out, (1, 2, 0)).astype(means2d.dtype)
